# baseline (speedup 1.0000x reference)
import functools

import jax
import jax.numpy as jnp
from jax import lax
from jax.experimental import pallas as pl
from jax.experimental.pallas import tpu as pltpu
from jax.experimental.pallas import tpu_sc as plsc

N_REF = 1_000_000
D_IN = 128
D_PC = 32
KTOP = 16
LANES = 16
NWORKERS = 32
BLK = 8192
NBLK = (N_REF + BLK - 1) // BLK
N_PAD = NBLK * BLK
WORDS_PER_TILE = N_PAD // NWORKERS
TILE_BATCHES = WORDS_PER_TILE // LANES


def _dist_body(din_ref, tm_ref, ref_ref, o_ref):
    q = jnp.dot(din_ref[...], tm_ref[...],
                preferred_element_type=jnp.float32)
    q4 = jnp.concatenate([q, q, q, q], axis=1)
    x = ref_ref[...]
    diff = jnp.abs(x - q4)
    s = jnp.sum(diff.reshape(BLK // 4, 4, D_PC), axis=2)
    d = s.reshape(BLK)
    rows = pl.program_id(0) * BLK + lax.broadcasted_iota(jnp.int32, (BLK,), 0)
    o_ref[...] = jnp.where(rows < N_REF, d, jnp.inf)


_dist_tc = pl.pallas_call(
    _dist_body,
    grid=(NBLK,),
    in_specs=[
        pl.BlockSpec((1, D_IN), lambda b: (0, 0)),
        pl.BlockSpec((D_IN, D_PC), lambda b: (0, 0)),
        pl.BlockSpec((BLK // 4, D_IN), lambda b: (b, 0)),
    ],
    out_specs=pl.BlockSpec((BLK,), lambda b: (b,)),
    out_shape=jax.ShapeDtypeStruct((N_PAD,), jnp.float32),
)


def _merge_top16(td, tp, nd, np_):
    sd, sp = plsc.sort_key_val(nd, np_, descending=True)
    take = td <= sd
    ld = jnp.where(take, td, sd)
    lp = jnp.where(take, tp, sp)
    od, op = plsc.sort_key_val(ld, lp)
    return od, op


def _topk_body(d_hbm, p_hbm, outd_hbm, outp_hbm, d_v, p_v, res_v, sem):
    cid = lax.axis_index("c")
    sid = lax.axis_index("s")
    wid = cid * 16 + sid
    base = wid * WORDS_PER_TILE

    H = WORDS_PER_TILE // 2
    pltpu.async_copy(d_hbm.at[pl.ds(base, H)], d_v.at[pl.ds(0, H)], sem)
    pltpu.async_copy(d_hbm.at[pl.ds(base + H, H)], d_v.at[pl.ds(H, H)], sem)
    pltpu.async_copy(p_hbm.at[pl.ds(base, H)], p_v.at[pl.ds(0, H)], sem)
    pltpu.async_copy(p_hbm.at[pl.ds(base + H, H)], p_v.at[pl.ds(H, H)], sem)
    for q in range(4):
        pltpu.make_async_copy(d_hbm.at[pl.ds(0, H)],
                              d_v.at[pl.ds(0, H)], sem).wait()

    def bstep(b, c2):
        topd, topp, thr = c2
        dv = d_v[pl.ds(b * LANES, LANES)]

        def merge(c3):
            pv = p_v[pl.ds(b * LANES, LANES)]
            sd, sp = _merge_top16(c3[0], c3[1], dv, pv)
            return (sd, sp, jnp.max(sd))

        return lax.cond(jnp.any(dv < thr), merge, lambda c3: c3, c2)

    carry = (jnp.full((LANES,), jnp.inf, jnp.float32),
             jnp.zeros((LANES,), jnp.float32),
             jnp.array(jnp.inf, jnp.float32))
    carry = lax.fori_loop(0, TILE_BATCHES, bstep, carry)

    res_v[pl.ds(0, 16)] = carry[0]
    res_v[pl.ds(16, 16)] = carry[1]
    pltpu.sync_copy(res_v.at[pl.ds(0, 16)], outd_hbm.at[pl.ds(wid * 16, 16)])
    pltpu.sync_copy(res_v.at[pl.ds(16, 16)], outp_hbm.at[pl.ds(wid * 16, 16)])


_topk_sc = functools.partial(
    pl.kernel,
    out_type=[jax.ShapeDtypeStruct((NWORKERS * 16,), jnp.float32),
              jax.ShapeDtypeStruct((NWORKERS * 16,), jnp.float32)],
    mesh=plsc.VectorSubcoreMesh(core_axis_name="c", subcore_axis_name="s"),
    compiler_params=pltpu.CompilerParams(needs_layout_passes=False),
    scratch_types=[
        pltpu.VMEM((WORDS_PER_TILE,), jnp.float32),
        pltpu.VMEM((WORDS_PER_TILE,), jnp.float32),
        pltpu.VMEM((32,), jnp.float32),
        pltpu.SemaphoreType.DMA,
    ],
)(_topk_body)


def _merge_tc_body(d_ref, p_ref, o_ref):
    d = d_ref[...]
    p = p_ref[...]
    ii = (lax.broadcasted_iota(jnp.int32, (4, 128), 0) * 128
          + lax.broadcasted_iota(jnp.int32, (4, 128), 1))

    def step(t, carry):
        s, dd = carry
        m = jnp.min(dd)
        eq = dd == m
        idx = jnp.min(jnp.where(eq, ii, jnp.int32(1 << 30)))
        sel = ii == idx
        s = s + jnp.sum(jnp.where(sel, p, 0.0))
        dd = jnp.where(sel, jnp.inf, dd)
        return (s, dd)

    s, _ = lax.fori_loop(0, KTOP, step, (jnp.float32(0.0), d))
    o_ref[0, 0] = s * (1.0 / KTOP)


_merge_tc = pl.pallas_call(
    _merge_tc_body,
    out_shape=jax.ShapeDtypeStruct((1, 1), jnp.float32),
    out_specs=pl.BlockSpec(memory_space=pltpu.SMEM),
)


def kernel(data_in, pseudotimes_arr, ref_data, transform_mat, K):
    del K
    dists = _dist_tc(data_in, transform_mat,
                     ref_data.reshape(N_REF // 4, D_IN))
    pts_p = jnp.pad(pseudotimes_arr, (0, N_PAD - N_REF))
    topd, topp = _topk_sc(dists, pts_p)
    merged = _merge_tc(topd.reshape(4, 128), topp.reshape(4, 128))
    return merged.reshape(1)


# speedup vs baseline: 1.3329x; 1.3329x over previous
import functools

import jax
import jax.numpy as jnp
from jax import lax
from jax.experimental import pallas as pl
from jax.experimental.pallas import tpu as pltpu
from jax.experimental.pallas import tpu_sc as plsc

N_REF = 1_000_000
D_IN = 128
D_PC = 32
KTOP = 16
LANES = 16
NWORKERS = 32

TAIL_ROWS = 458_752
HEAD_ROWS = N_REF - TAIL_ROWS
TAIL_ROW0 = HEAD_ROWS

BLK = 8192
NBLK = (HEAD_ROWS + BLK - 1) // BLK
N_PAD = NBLK * BLK
WORDS_PER_TILE = N_PAD // NWORKERS
TILE_BATCHES = WORDS_PER_TILE // LANES

T_BATCHES_PER_W = TAIL_ROWS // NWORKERS // LANES
T_CHUNK_BATCHES = 64
T_CHUNK_ROWS = T_CHUNK_BATCHES * LANES
T_CHUNKS = T_BATCHES_PER_W // T_CHUNK_BATCHES
T_ROWS_PER_W = T_BATCHES_PER_W * LANES
NSPLIT = 4
SUB_W = T_CHUNK_ROWS * D_PC // NSPLIT


def _dist_body(din_ref, tm_ref, ref_ref, o_ref):
    q = jnp.dot(din_ref[...], tm_ref[...],
                preferred_element_type=jnp.float32)
    x = ref_ref[...]
    d = jnp.sum(jnp.abs(x - q), axis=1)
    rows = pl.program_id(0) * BLK + lax.broadcasted_iota(jnp.int32, (BLK,), 0)
    o_ref[...] = jnp.where(rows < HEAD_ROWS, d, jnp.inf)


_dist_tc = pl.pallas_call(
    _dist_body,
    grid=(NBLK,),
    in_specs=[
        pl.BlockSpec((1, D_IN), lambda b: (0, 0)),
        pl.BlockSpec((D_IN, D_PC), lambda b: (0, 0)),
        pl.BlockSpec((BLK, D_PC), lambda b: (b, 0)),
    ],
    out_specs=pl.BlockSpec((BLK,), lambda b: (b,)),
    out_shape=jax.ShapeDtypeStruct((N_PAD,), jnp.float32),
)


def _merge_top16(td, tp, nd, np_):
    sd, sp = plsc.sort_key_val(nd, np_, descending=True)
    take = td <= sd
    ld = jnp.where(take, td, sd)
    lp = jnp.where(take, tp, sp)
    od, op = plsc.sort_key_val(ld, lp)
    return od, op


def _topk_body(d_hbm, p_hbm, outd_hbm, outp_hbm, d_v, p_v, res_v, sem):
    cid = lax.axis_index("c")
    sid = lax.axis_index("s")
    wid = cid * 16 + sid
    base = wid * WORDS_PER_TILE

    H = WORDS_PER_TILE // 2
    pltpu.async_copy(d_hbm.at[pl.ds(base, H)], d_v.at[pl.ds(0, H)], sem)
    pltpu.async_copy(d_hbm.at[pl.ds(base + H, H)], d_v.at[pl.ds(H, H)], sem)
    pltpu.async_copy(p_hbm.at[pl.ds(base, H)], p_v.at[pl.ds(0, H)], sem)
    pltpu.async_copy(p_hbm.at[pl.ds(base + H, H)], p_v.at[pl.ds(H, H)], sem)
    for _ in range(4):
        pltpu.make_async_copy(d_hbm.at[pl.ds(0, H)],
                              d_v.at[pl.ds(0, H)], sem).wait()

    def bstep(b, c2):
        topd, topp, thr = c2
        dv = d_v[pl.ds(b * LANES, LANES)]

        def merge(c3):
            pv = p_v[pl.ds(b * LANES, LANES)]
            sd, sp = _merge_top16(c3[0], c3[1], dv, pv)
            return (sd, sp, jnp.max(sd))

        return lax.cond(jnp.any(dv < thr), merge, lambda c3: c3, c2)

    carry = (jnp.full((LANES,), jnp.inf, jnp.float32),
             jnp.zeros((LANES,), jnp.float32),
             jnp.array(jnp.inf, jnp.float32))
    carry = lax.fori_loop(0, TILE_BATCHES, bstep, carry)

    res_v[pl.ds(0, 16)] = carry[0]
    res_v[pl.ds(16, 16)] = carry[1]
    pltpu.sync_copy(res_v.at[pl.ds(0, 16)], outd_hbm.at[pl.ds(wid * 16, 16)])
    pltpu.sync_copy(res_v.at[pl.ds(16, 16)], outp_hbm.at[pl.ds(wid * 16, 16)])


_topk_sc = functools.partial(
    pl.kernel,
    out_type=[jax.ShapeDtypeStruct((NWORKERS * 16,), jnp.float32),
              jax.ShapeDtypeStruct((NWORKERS * 16,), jnp.float32)],
    mesh=plsc.VectorSubcoreMesh(core_axis_name="c", subcore_axis_name="s"),
    compiler_params=pltpu.CompilerParams(needs_layout_passes=False),
    scratch_types=[
        pltpu.VMEM((WORDS_PER_TILE,), jnp.float32),
        pltpu.VMEM((WORDS_PER_TILE,), jnp.float32),
        pltpu.VMEM((32,), jnp.float32),
        pltpu.SemaphoreType.DMA,
    ],
)(_topk_body)


def _tail_body(din_hbm, pt_hbm, ref_flat, tm_hbm, outd_hbm, outp_hbm,
               din_v, tm_v, ref_v0, ref_v1, pt_v0, pt_v1, res_v,
               sem_r0, sem_r1, sem_p0, sem_p1):
    cid = lax.axis_index("c")
    sid = lax.axis_index("s")
    wid = cid * 16 + sid

    pltpu.sync_copy(din_hbm, din_v)
    pltpu.sync_copy(tm_hbm, tm_v)

    def qstep(j, qc):
        q0, q1 = qc
        dv = din_v[0, pl.ds(j * LANES, LANES)]
        for l in range(LANES):
            s = dv[l]
            row = j * LANES + l
            q0 = q0 + s * tm_v[row, pl.ds(0, 16)]
            q1 = q1 + s * tm_v[row, pl.ds(16, 16)]
        return (q0, q1)

    z16 = jnp.zeros((LANES,), jnp.float32)
    q0, q1 = lax.fori_loop(0, D_IN // LANES, qstep, (z16, z16))
    qs = tuple(q0[d] for d in range(16)) + tuple(q1[d] for d in range(16))

    iota = lax.broadcasted_iota(jnp.int32, (LANES,), 0)
    iota32 = iota * D_PC

    def process_batches(ref_c, pt_c, nbatches, carry):
        def bstep(b, c2):
            topd, topp, thr = c2
            ridx = b * (LANES * D_PC) + iota32
            acc = None
            for d in range(D_PC):
                v = plsc.load_gather(ref_c, [ridx + d])
                ad = jnp.abs(v - qs[d])
                acc = ad if acc is None else acc + ad
            pts = pt_c[pl.ds(b * LANES, LANES)]

            def merge(c3):
                sd, sp = _merge_top16(c3[0], c3[1], acc, pts)
                return (sd, sp, jnp.max(sd))

            return lax.cond(jnp.any(acc < thr), merge, lambda c3: c3, c2)

        return lax.fori_loop(0, nbatches, bstep, carry)

    lrow0 = wid * T_ROWS_PER_W

    def start_chunk(c, rv, pv, sem_r, sem_p):
        r0 = lrow0 + c * T_CHUNK_ROWS
        base = r0 * D_PC
        for q in range(NSPLIT):
            pltpu.async_copy(ref_flat.at[pl.ds(base + q * SUB_W, SUB_W)],
                             rv.at[pl.ds(q * SUB_W, SUB_W)], sem_r)
        pltpu.async_copy(pt_hbm.at[pl.ds(TAIL_ROW0 + r0, T_CHUNK_ROWS)],
                         pv, sem_p)

    def wait_chunk(rv, pv, sem_r, sem_p):
        for q in range(NSPLIT):
            pltpu.make_async_copy(ref_flat.at[pl.ds(0, SUB_W)],
                                  rv.at[pl.ds(q * SUB_W, SUB_W)],
                                  sem_r).wait()
        pltpu.make_async_copy(pt_hbm.at[pl.ds(0, T_CHUNK_ROWS)], pv,
                              sem_p).wait()

    carry = (jnp.full((LANES,), jnp.inf, jnp.float32),
             jnp.zeros((LANES,), jnp.float32),
             jnp.array(jnp.inf, jnp.float32))

    start_chunk(0, ref_v0, pt_v0, sem_r0, sem_p0)

    def cstep(c, carry):
        def even(carry):
            @pl.when(c + 1 < T_CHUNKS)
            def _():
                start_chunk(c + 1, ref_v1, pt_v1, sem_r1, sem_p1)
            wait_chunk(ref_v0, pt_v0, sem_r0, sem_p0)
            return process_batches(ref_v0, pt_v0, T_CHUNK_BATCHES, carry)

        def odd(carry):
            @pl.when(c + 1 < T_CHUNKS)
            def _():
                start_chunk(c + 1, ref_v0, pt_v0, sem_r0, sem_p0)
            wait_chunk(ref_v1, pt_v1, sem_r1, sem_p1)
            return process_batches(ref_v1, pt_v1, T_CHUNK_BATCHES, carry)

        return lax.cond(c % 2 == 0, even, odd, carry)

    carry = lax.fori_loop(0, T_CHUNKS, cstep, carry)

    res_v[pl.ds(0, 16)] = carry[0]
    res_v[pl.ds(16, 16)] = carry[1]
    pltpu.sync_copy(res_v.at[pl.ds(0, 16)], outd_hbm.at[pl.ds(wid * 16, 16)])
    pltpu.sync_copy(res_v.at[pl.ds(16, 16)], outp_hbm.at[pl.ds(wid * 16, 16)])


_tail_sc = functools.partial(
    pl.kernel,
    out_type=[jax.ShapeDtypeStruct((NWORKERS * 16,), jnp.float32),
              jax.ShapeDtypeStruct((NWORKERS * 16,), jnp.float32)],
    mesh=plsc.VectorSubcoreMesh(core_axis_name="c", subcore_axis_name="s"),
    compiler_params=pltpu.CompilerParams(needs_layout_passes=False),
    scratch_types=[
        pltpu.VMEM((1, D_IN), jnp.float32),
        pltpu.VMEM((D_IN, D_PC), jnp.float32),
        pltpu.VMEM((T_CHUNK_ROWS * D_PC,), jnp.float32),
        pltpu.VMEM((T_CHUNK_ROWS * D_PC,), jnp.float32),
        pltpu.VMEM((T_CHUNK_ROWS,), jnp.float32),
        pltpu.VMEM((T_CHUNK_ROWS,), jnp.float32),
        pltpu.VMEM((32,), jnp.float32),
        pltpu.SemaphoreType.DMA,
        pltpu.SemaphoreType.DMA,
        pltpu.SemaphoreType.DMA,
        pltpu.SemaphoreType.DMA,
    ],
)(_tail_body)


def _merge_tc_body(d1_ref, d2_ref, p1_ref, p2_ref, o_ref):
    d = jnp.concatenate([d1_ref[...], d2_ref[...]], axis=0)
    p = jnp.concatenate([p1_ref[...], p2_ref[...]], axis=0)
    ii = (lax.broadcasted_iota(jnp.int32, (8, 128), 0) * 128
          + lax.broadcasted_iota(jnp.int32, (8, 128), 1))

    def step(t, carry):
        s, dd = carry
        m = jnp.min(dd)
        eq = dd == m
        idx = jnp.min(jnp.where(eq, ii, jnp.int32(1 << 30)))
        sel = ii == idx
        s = s + jnp.sum(jnp.where(sel, p, 0.0))
        dd = jnp.where(sel, jnp.inf, dd)
        return (s, dd)

    s, _ = lax.fori_loop(0, KTOP, step, (jnp.float32(0.0), d))
    o_ref[0, 0] = s * (1.0 / KTOP)


_merge_tc = pl.pallas_call(
    _merge_tc_body,
    out_shape=jax.ShapeDtypeStruct((1, 1), jnp.float32),
    out_specs=pl.BlockSpec(memory_space=pltpu.SMEM),
)


def kernel(data_in, pseudotimes_arr, ref_data, transform_mat, K):
    del K
    ref_tail_flat = lax.slice(ref_data, (TAIL_ROW0, 0),
                              (N_REF, D_PC)).reshape(TAIL_ROWS * D_PC)
    taild, tailp = _tail_sc(data_in, pseudotimes_arr, ref_tail_flat,
                            transform_mat)
    dists = _dist_tc(data_in, transform_mat, ref_data)
    headd, headp = _topk_sc(dists, pseudotimes_arr)
    merged = _merge_tc(headd.reshape(4, 128), taild.reshape(4, 128),
                       headp.reshape(4, 128), tailp.reshape(4, 128))
    return merged.reshape(1)


# speedup vs baseline: 1.4590x; 1.0946x over previous
import functools

import jax
import jax.numpy as jnp
from jax import lax
from jax.experimental import pallas as pl
from jax.experimental.pallas import tpu as pltpu
from jax.experimental.pallas import tpu_sc as plsc

N_REF = 1_000_000
D_IN = 128
D_PC = 32
KTOP = 16
LANES = 16
NWORKERS = 32

TAIL_ROWS = 458_752
HEAD_ROWS = N_REF - TAIL_ROWS
TAIL_ROW0 = HEAD_ROWS

BLK = 8192
NBLK = (HEAD_ROWS + BLK - 1) // BLK
N_PAD = NBLK * BLK
WORDS_PER_TILE = N_PAD // NWORKERS
TILE_BATCHES = WORDS_PER_TILE // LANES

T_BATCHES_PER_W = TAIL_ROWS // NWORKERS // LANES
T_CHUNK_BATCHES = 64
T_CHUNK_ROWS = T_CHUNK_BATCHES * LANES
T_CHUNKS = T_BATCHES_PER_W // T_CHUNK_BATCHES
T_ROWS_PER_W = T_BATCHES_PER_W * LANES
NSPLIT = 4
SUB_W = T_CHUNK_ROWS * D_PC // NSPLIT


def _dist_body(din_ref, tm_ref, ref_ref, o_ref):
    q = jnp.dot(din_ref[...], tm_ref[...],
                preferred_element_type=jnp.float32)
    x = ref_ref[...]
    d = jnp.sum(jnp.abs(x - q), axis=1)
    rows = pl.program_id(0) * BLK + lax.broadcasted_iota(jnp.int32, (BLK,), 0)
    o_ref[...] = jnp.where(rows < HEAD_ROWS, d, jnp.inf)


_dist_tc = pl.pallas_call(
    _dist_body,
    grid=(NBLK,),
    in_specs=[
        pl.BlockSpec((1, D_IN), lambda b: (0, 0)),
        pl.BlockSpec((D_IN, D_PC), lambda b: (0, 0)),
        pl.BlockSpec((BLK, D_PC), lambda b: (b, 0)),
    ],
    out_specs=pl.BlockSpec((BLK,), lambda b: (b,)),
    out_shape=jax.ShapeDtypeStruct((N_PAD,), jnp.float32),
)


def _merge_top16(td, tp, nd, np_):
    sd, sp = plsc.sort_key_val(nd, np_, descending=True)
    take = td <= sd
    ld = jnp.where(take, td, sd)
    lp = jnp.where(take, tp, sp)
    od, op = plsc.sort_key_val(ld, lp)
    return od, op


def _topk_body(d_hbm, p_hbm, outd_hbm, outp_hbm, d_v, p_v, res_v, sem):
    cid = lax.axis_index("c")
    sid = lax.axis_index("s")
    wid = cid * 16 + sid
    base = wid * WORDS_PER_TILE

    H = WORDS_PER_TILE // 2
    pltpu.async_copy(d_hbm.at[pl.ds(base, H)], d_v.at[pl.ds(0, H)], sem)
    pltpu.async_copy(d_hbm.at[pl.ds(base + H, H)], d_v.at[pl.ds(H, H)], sem)
    pltpu.async_copy(p_hbm.at[pl.ds(base, H)], p_v.at[pl.ds(0, H)], sem)
    pltpu.async_copy(p_hbm.at[pl.ds(base + H, H)], p_v.at[pl.ds(H, H)], sem)
    for _ in range(4):
        pltpu.make_async_copy(d_hbm.at[pl.ds(0, H)],
                              d_v.at[pl.ds(0, H)], sem).wait()

    def bstep(b, c2):
        topd, topp, thr = c2
        dv = d_v[pl.ds(b * LANES, LANES)]

        def merge(c3):
            pv = p_v[pl.ds(b * LANES, LANES)]
            sd, sp = _merge_top16(c3[0], c3[1], dv, pv)
            return (sd, sp, jnp.max(sd))

        return lax.cond(jnp.any(dv < thr), merge, lambda c3: c3, c2)

    carry = (jnp.full((LANES,), jnp.inf, jnp.float32),
             jnp.zeros((LANES,), jnp.float32),
             jnp.array(jnp.inf, jnp.float32))
    carry = lax.fori_loop(0, TILE_BATCHES, bstep, carry)

    res_v[pl.ds(0, 16)] = carry[0]
    res_v[pl.ds(16, 16)] = carry[1]
    pltpu.sync_copy(res_v.at[pl.ds(0, 16)], outd_hbm.at[pl.ds(wid * 16, 16)])
    pltpu.sync_copy(res_v.at[pl.ds(16, 16)], outp_hbm.at[pl.ds(wid * 16, 16)])


_topk_sc = functools.partial(
    pl.kernel,
    out_type=[jax.ShapeDtypeStruct((NWORKERS * 16,), jnp.float32),
              jax.ShapeDtypeStruct((NWORKERS * 16,), jnp.float32)],
    mesh=plsc.VectorSubcoreMesh(core_axis_name="c", subcore_axis_name="s"),
    compiler_params=pltpu.CompilerParams(needs_layout_passes=False),
    scratch_types=[
        pltpu.VMEM((WORDS_PER_TILE,), jnp.float32),
        pltpu.VMEM((WORDS_PER_TILE,), jnp.float32),
        pltpu.VMEM((32,), jnp.float32),
        pltpu.SemaphoreType.DMA,
    ],
)(_topk_body)


def _tail_body(din_hbm, pt_hbm, ref_flat, tm_hbm, outd_hbm, outp_hbm,
               din_v, tm_v, ref_v0, ref_v1, pt_v0, pt_v1, res_v,
               sem_r0, sem_r1, sem_p0, sem_p1):
    cid = lax.axis_index("c")
    sid = lax.axis_index("s")
    wid = cid * 16 + sid

    pltpu.sync_copy(din_hbm, din_v)
    pltpu.sync_copy(tm_hbm, tm_v)

    def qstep(j, qc):
        q0, q1 = qc
        dv = din_v[0, pl.ds(j * LANES, LANES)]
        for l in range(LANES):
            s = dv[l]
            row = j * LANES + l
            q0 = q0 + s * tm_v[row, pl.ds(0, 16)]
            q1 = q1 + s * tm_v[row, pl.ds(16, 16)]
        return (q0, q1)

    z16 = jnp.zeros((LANES,), jnp.float32)
    q0, q1 = lax.fori_loop(0, D_IN // LANES, qstep, (z16, z16))
    qs = tuple(q0[d] for d in range(16)) + tuple(q1[d] for d in range(16))

    iota = lax.broadcasted_iota(jnp.int32, (LANES,), 0)
    iota32 = iota * D_PC

    def process_batches(ref_c, pt_c, nbatches, carry):
        def bstep(b, c2):
            topd, topp, thr = c2
            ridx = b * (LANES * D_PC) + iota32
            acc = None
            for d in range(D_PC):
                v = plsc.load_gather(ref_c, [ridx + d])
                ad = jnp.abs(v - qs[d])
                acc = ad if acc is None else acc + ad
            pts = pt_c[pl.ds(b * LANES, LANES)]

            def merge(c3):
                sd, sp = _merge_top16(c3[0], c3[1], acc, pts)
                return (sd, sp, jnp.max(sd))

            return lax.cond(jnp.any(acc < thr), merge, lambda c3: c3, c2)

        return lax.fori_loop(0, nbatches, bstep, carry)

    lrow0 = wid * T_ROWS_PER_W

    def start_chunk(c, rv, pv, sem_r, sem_p):
        r0 = lrow0 + c * T_CHUNK_ROWS
        base = r0 * D_PC
        for q in range(NSPLIT):
            pltpu.async_copy(ref_flat.at[pl.ds(base + q * SUB_W, SUB_W)],
                             rv.at[pl.ds(q * SUB_W, SUB_W)], sem_r)
        pltpu.async_copy(pt_hbm.at[pl.ds(TAIL_ROW0 + r0, T_CHUNK_ROWS)],
                         pv, sem_p)

    def wait_chunk(rv, pv, sem_r, sem_p):
        for q in range(NSPLIT):
            pltpu.make_async_copy(ref_flat.at[pl.ds(0, SUB_W)],
                                  rv.at[pl.ds(q * SUB_W, SUB_W)],
                                  sem_r).wait()
        pltpu.make_async_copy(pt_hbm.at[pl.ds(0, T_CHUNK_ROWS)], pv,
                              sem_p).wait()

    carry = (jnp.full((LANES,), jnp.inf, jnp.float32),
             jnp.zeros((LANES,), jnp.float32),
             jnp.array(jnp.inf, jnp.float32))

    start_chunk(0, ref_v0, pt_v0, sem_r0, sem_p0)

    def cstep(c, carry):
        def even(carry):
            @pl.when(c + 1 < T_CHUNKS)
            def _():
                start_chunk(c + 1, ref_v1, pt_v1, sem_r1, sem_p1)
            wait_chunk(ref_v0, pt_v0, sem_r0, sem_p0)
            return process_batches(ref_v0, pt_v0, T_CHUNK_BATCHES, carry)

        def odd(carry):
            @pl.when(c + 1 < T_CHUNKS)
            def _():
                start_chunk(c + 1, ref_v0, pt_v0, sem_r0, sem_p0)
            wait_chunk(ref_v1, pt_v1, sem_r1, sem_p1)
            return process_batches(ref_v1, pt_v1, T_CHUNK_BATCHES, carry)

        return lax.cond(c % 2 == 0, even, odd, carry)

    carry = lax.fori_loop(0, T_CHUNKS, cstep, carry)

    res_v[pl.ds(0, 16)] = carry[0]
    res_v[pl.ds(16, 16)] = carry[1]
    pltpu.sync_copy(res_v.at[pl.ds(0, 16)], outd_hbm.at[pl.ds(wid * 16, 16)])
    pltpu.sync_copy(res_v.at[pl.ds(16, 16)], outp_hbm.at[pl.ds(wid * 16, 16)])


_tail_sc = functools.partial(
    pl.kernel,
    out_type=[jax.ShapeDtypeStruct((NWORKERS * 16,), jnp.float32),
              jax.ShapeDtypeStruct((NWORKERS * 16,), jnp.float32)],
    mesh=plsc.VectorSubcoreMesh(core_axis_name="c", subcore_axis_name="s"),
    compiler_params=pltpu.CompilerParams(needs_layout_passes=False),
    scratch_types=[
        pltpu.VMEM((1, D_IN), jnp.float32),
        pltpu.VMEM((D_IN, D_PC), jnp.float32),
        pltpu.VMEM((T_CHUNK_ROWS * D_PC,), jnp.float32),
        pltpu.VMEM((T_CHUNK_ROWS * D_PC,), jnp.float32),
        pltpu.VMEM((T_CHUNK_ROWS,), jnp.float32),
        pltpu.VMEM((T_CHUNK_ROWS,), jnp.float32),
        pltpu.VMEM((32,), jnp.float32),
        pltpu.SemaphoreType.DMA,
        pltpu.SemaphoreType.DMA,
        pltpu.SemaphoreType.DMA,
        pltpu.SemaphoreType.DMA,
    ],
)(_tail_body)


def _merge_tc_body(d1_ref, d2_ref, p1_ref, p2_ref, o_ref):
    d = jnp.concatenate([d1_ref[...], d2_ref[...]], axis=0)
    p = jnp.concatenate([p1_ref[...], p2_ref[...]], axis=0)
    ii = (lax.broadcasted_iota(jnp.int32, (8, 128), 0) * 128
          + lax.broadcasted_iota(jnp.int32, (8, 128), 1))

    def step(t, carry):
        s, dd = carry
        m = jnp.min(dd)
        eq = dd == m
        idx = jnp.min(jnp.where(eq, ii, jnp.int32(1 << 30)))
        sel = ii == idx
        s = s + jnp.sum(jnp.where(sel, p, 0.0))
        dd = jnp.where(sel, jnp.inf, dd)
        return (s, dd)

    s, _ = lax.fori_loop(0, KTOP, step, (jnp.float32(0.0), d))
    o_ref[0, 0] = s * (1.0 / KTOP)


_merge_tc = pl.pallas_call(
    _merge_tc_body,
    out_shape=jax.ShapeDtypeStruct((1, 1), jnp.float32),
    out_specs=pl.BlockSpec(memory_space=pltpu.SMEM),
)


def kernel(data_in, pseudotimes_arr, ref_data, transform_mat, K):
    del K
    ref_head = lax.slice(ref_data, (0, 0), (N_PAD, D_PC))
    ref_tail_flat = lax.slice(ref_data, (TAIL_ROW0, 0),
                              (N_REF, D_PC)).reshape(TAIL_ROWS * D_PC)
    dists = _dist_tc(data_in, transform_mat, ref_head)
    taild, tailp = _tail_sc(data_in, pseudotimes_arr, ref_tail_flat,
                            transform_mat)
    headd, headp = _topk_sc(dists, pseudotimes_arr)
    merged = _merge_tc(headd.reshape(4, 128), taild.reshape(4, 128),
                       headp.reshape(4, 128), tailp.reshape(4, 128))
    return merged.reshape(1)


# speedup vs baseline: 1.8281x; 1.2530x over previous
import functools

import jax
import jax.numpy as jnp
from jax import lax
from jax.experimental import pallas as pl
from jax.experimental.pallas import tpu as pltpu
from jax.experimental.pallas import tpu_sc as plsc

N_REF = 1_000_000
D_IN = 128
D_PC = 32
KTOP = 16
LANES = 16
NWORKERS = 32
BLK = 8192
NBLK = (N_REF + BLK - 1) // BLK
N_PAD = NBLK * BLK
WORDS_PER_TILE = N_PAD // NWORKERS
TILE_BATCHES = WORDS_PER_TILE // LANES


def _dist_body(din_ref, tm_ref, ref_ref, o_ref):
    q = jnp.dot(din_ref[...], tm_ref[...],
                preferred_element_type=jnp.float32)
    x = ref_ref[...]
    d = jnp.sum(jnp.abs(x - q), axis=1)
    rows = pl.program_id(0) * BLK + lax.broadcasted_iota(jnp.int32, (BLK,), 0)
    o_ref[...] = jnp.where(rows < N_REF, d, jnp.inf)


_dist_tc = pl.pallas_call(
    _dist_body,
    grid=(NBLK,),
    in_specs=[
        pl.BlockSpec((1, D_IN), lambda b: (0, 0)),
        pl.BlockSpec((D_IN, D_PC), lambda b: (0, 0)),
        pl.BlockSpec((BLK, D_PC), lambda b: (b, 0)),
    ],
    out_specs=pl.BlockSpec((BLK,), lambda b: (b,)),
    out_shape=jax.ShapeDtypeStruct((N_PAD,), jnp.float32),
)


def _merge_top16(td, tp, nd, np_):
    sd, sp = plsc.sort_key_val(nd, np_, descending=True)
    take = td <= sd
    ld = jnp.where(take, td, sd)
    lp = jnp.where(take, tp, sp)
    od, op = plsc.sort_key_val(ld, lp)
    return od, op


def _topk_body(d_hbm, p_hbm, outd_hbm, outp_hbm, d_v, p_v, res_v, sem):
    cid = lax.axis_index("c")
    sid = lax.axis_index("s")
    wid = cid * 16 + sid
    base = wid * WORDS_PER_TILE

    H = WORDS_PER_TILE // 2
    pltpu.async_copy(d_hbm.at[pl.ds(base, H)], d_v.at[pl.ds(0, H)], sem)
    pltpu.async_copy(d_hbm.at[pl.ds(base + H, H)], d_v.at[pl.ds(H, H)], sem)
    pltpu.async_copy(p_hbm.at[pl.ds(base, H)], p_v.at[pl.ds(0, H)], sem)
    pltpu.async_copy(p_hbm.at[pl.ds(base + H, H)], p_v.at[pl.ds(H, H)], sem)
    for q in range(4):
        pltpu.make_async_copy(d_hbm.at[pl.ds(0, H)],
                              d_v.at[pl.ds(0, H)], sem).wait()

    def bstep(b, c2):
        topd, topp, thr = c2
        dv = d_v[pl.ds(b * LANES, LANES)]

        def merge(c3):
            pv = p_v[pl.ds(b * LANES, LANES)]
            sd, sp = _merge_top16(c3[0], c3[1], dv, pv)
            return (sd, sp, jnp.max(sd))

        return lax.cond(jnp.any(dv < thr), merge, lambda c3: c3, c2)

    carry = (jnp.full((LANES,), jnp.inf, jnp.float32),
             jnp.zeros((LANES,), jnp.float32),
             jnp.array(jnp.inf, jnp.float32))
    carry = lax.fori_loop(0, TILE_BATCHES, bstep, carry)

    res_v[pl.ds(0, 16)] = carry[0]
    res_v[pl.ds(16, 16)] = carry[1]
    pltpu.sync_copy(res_v.at[pl.ds(0, 16)], outd_hbm.at[pl.ds(wid * 16, 16)])
    pltpu.sync_copy(res_v.at[pl.ds(16, 16)], outp_hbm.at[pl.ds(wid * 16, 16)])


_topk_sc = functools.partial(
    pl.kernel,
    out_type=[jax.ShapeDtypeStruct((NWORKERS * 16,), jnp.float32),
              jax.ShapeDtypeStruct((NWORKERS * 16,), jnp.float32)],
    mesh=plsc.VectorSubcoreMesh(core_axis_name="c", subcore_axis_name="s"),
    compiler_params=pltpu.CompilerParams(needs_layout_passes=False),
    scratch_types=[
        pltpu.VMEM((WORDS_PER_TILE,), jnp.float32),
        pltpu.VMEM((WORDS_PER_TILE,), jnp.float32),
        pltpu.VMEM((32,), jnp.float32),
        pltpu.SemaphoreType.DMA,
    ],
)(_topk_body)


def _merge_tc_body(d_ref, p_ref, o_ref):
    d = d_ref[...]
    p = p_ref[...]
    ii = (lax.broadcasted_iota(jnp.int32, (4, 128), 0) * 128
          + lax.broadcasted_iota(jnp.int32, (4, 128), 1))

    def step(t, carry):
        s, dd = carry
        m = jnp.min(dd)
        eq = dd == m
        idx = jnp.min(jnp.where(eq, ii, jnp.int32(1 << 30)))
        sel = ii == idx
        s = s + jnp.sum(jnp.where(sel, p, 0.0))
        dd = jnp.where(sel, jnp.inf, dd)
        return (s, dd)

    s, _ = lax.fori_loop(0, KTOP, step, (jnp.float32(0.0), d))
    o_ref[0, 0] = s * (1.0 / KTOP)


_merge_tc = pl.pallas_call(
    _merge_tc_body,
    out_shape=jax.ShapeDtypeStruct((1, 1), jnp.float32),
    out_specs=pl.BlockSpec(memory_space=pltpu.SMEM),
)


def kernel(data_in, pseudotimes_arr, ref_data, transform_mat, K):
    del K
    dists = _dist_tc(data_in, transform_mat, ref_data)
    pts_p = jnp.pad(pseudotimes_arr, (0, N_PAD - N_REF))
    topd, topp = _topk_sc(dists, pts_p)
    merged = _merge_tc(topd.reshape(4, 128), topp.reshape(4, 128))
    return merged.reshape(1)
